# baseline (speedup 1.0000x reference)
import functools

import jax
import jax.numpy as jnp
from jax.experimental import pallas as pl
from jax.experimental.pallas import tpu as pltpu
from jax.experimental.pallas import tpu_sc as plsc

B, S, D = 1, 2048, 1024
H, KVH, HD = 16, 4, 64
E, FF = 8, 1024
T = B * S
QKW = (H + KVH) * HD
QKVW = (H + 2 * KVH) * HD

BM_A = 256
BQ = 512
BM_C = 256
BM_M = 512
NB_M = T // BM_M

_BF = jnp.bfloat16
_F32 = jnp.float32


def _preattn_kernel(x_ref, w_ref, cos_ref, sin_ref, win_ref, qkw_ref,
                    q_ref, k_ref, v_ref):
    x = x_ref[...]
    h = x * jax.lax.rsqrt(jnp.mean(x * x, axis=-1, keepdims=True) + 1e-5)
    h = h * win_ref[...]
    qkv = jnp.dot(h.astype(_BF), w_ref[...], preferred_element_type=_F32)
    base = qkv[:, :QKW]
    vpart = qkv[:, QKW:QKVW]
    lane = jax.lax.broadcasted_iota(jnp.int32, (BM_A, QKW), 1)
    rot = jnp.where(lane % 2 == 0,
                    -pltpu.roll(base, QKW - 1, 1),
                    pltpu.roll(base, 1, 1))
    cosT = jnp.concatenate([cos_ref[...]] * (H + KVH), axis=1)
    sinT = jnp.concatenate([sin_ref[...]] * (H + KVH), axis=1)
    qk = base * cosT + rot * sinT
    lane = jax.lax.broadcasted_iota(jnp.int32, (QKW, H + KVH), 0)
    head = jax.lax.broadcasted_iota(jnp.int32, (QKW, H + KVH), 1)
    G = (lane // HD == head).astype(_F32)
    laneT = jax.lax.broadcasted_iota(jnp.int32, (H + KVH, QKW), 1)
    headT = jax.lax.broadcasted_iota(jnp.int32, (H + KVH, QKW), 0)
    GT = (laneT // HD == headT).astype(_F32)
    ms = jnp.dot(qk * qk, G, preferred_element_type=_F32) / HD
    bc = jnp.dot(jax.lax.rsqrt(ms + 1e-6), GT, preferred_element_type=_F32)
    qkn = qk * bc * qkw_ref[...]
    q_ref[...] = (qkn[:, : H * HD] * (HD ** -0.5)).astype(_BF)
    k_ref[...] = qkn[:, H * HD:].astype(_BF)
    v_ref[...] = vpart.astype(_BF)


def _attn_kernel(q_ref, k_ref, v_ref, o_ref):
    i = pl.program_id(1)
    GQ = H // KVH
    q = q_ref[0].reshape(GQ * BQ, HD)

    def chunk(j, acc, masked):
        kc = k_ref[0, pl.ds(j * BQ, BQ), :]
        s = jax.lax.dot_general(q, kc, (((1,), (1,)), ((), ())),
                                preferred_element_type=_F32)
        if masked:
            r = jax.lax.broadcasted_iota(jnp.int32, (GQ * BQ, BQ), 0)
            c = jax.lax.broadcasted_iota(jnp.int32, (GQ * BQ, BQ), 1)
            s = jnp.where(c <= r % BQ, s, -1e30)
        p = jnp.exp(s).astype(_BF)
        vc = v_ref[0, pl.ds(j * BQ, BQ), :]
        return acc + jnp.dot(p, vc, preferred_element_type=_F32)

    init = jnp.zeros((GQ * BQ, HD + 1), _F32)
    acc = jax.lax.fori_loop(0, i, lambda j, c: chunk(j, c, False), init)
    acc = chunk(i, acc, True)
    out = acc[:, :HD] / acc[:, HD:HD + 1]
    o_ref[0] = out.reshape(GQ, BQ, HD).astype(_BF)


def _postattn_kernel(a_ref, wo_ref, x_ref, wpost_ref, wrt_ref,
                     res_ref, h2_ref, ids_ref, iw_ref):
    ao = jnp.dot(a_ref[...], wo_ref[...], preferred_element_type=_F32)
    resid = x_ref[...] + ao
    res_ref[...] = resid
    h2 = resid * jax.lax.rsqrt(
        jnp.mean(resid * resid, axis=-1, keepdims=True) + 1e-5)
    h2 = h2 * wpost_ref[...]
    h2_ref[...] = h2
    logits = jnp.dot(h2, wrt_ref[...], preferred_element_type=_F32)
    mx = jnp.max(logits, axis=-1, keepdims=True)
    e_iota = jax.lax.broadcasted_iota(jnp.int32, logits.shape, 1)
    ids_ref[...] = jnp.min(jnp.where(logits == mx, e_iota, E),
                           axis=-1, keepdims=True)
    iw_ref[...] = jax.nn.sigmoid(mx)


def _expert_range(b, gs_ref):
    start = b * BM_M
    end = start + BM_M
    lo = jnp.zeros((), jnp.int32)
    cnt = jnp.zeros((), jnp.int32)
    for e in range(E):
        lo = lo + (gs_ref[e + 1] <= start).astype(jnp.int32)
        cnt = cnt + (gs_ref[e] < end).astype(jnp.int32)
    return lo, cnt - 1


def _moe_widx(b, j, gs_ref):
    lo, hi = _expert_range(b, gs_ref)
    return (jnp.clip(j - 1, lo, hi), 0, 0)


_FCH = 2


def _glu(x, w1_ref, w2_ref):
    fc = FF // _FCH
    y = jnp.zeros((x.shape[0], D), _F32)
    for f in range(_FCH):
        g = jnp.dot(x, w1_ref[:, f * fc:(f + 1) * fc].astype(_BF),
                    preferred_element_type=_F32)
        u = jnp.dot(x, w1_ref[:, FF + f * fc:FF + (f + 1) * fc].astype(_BF),
                    preferred_element_type=_F32)
        act = (jax.nn.silu(g) * u).astype(_BF)
        y = y + jnp.dot(act, w2_ref[f * fc:(f + 1) * fc, :].astype(_BF),
                        preferred_element_type=_F32)
    return y


def _moe_kernel(gs_ref, xp_ref, iw_ref, w1_ref, w2_ref,
                ws1_ref, ws2_ref, o_ref):
    b = pl.program_id(0)
    j = pl.program_id(1)

    @pl.when(j == 0)
    def _shared():
        x = xp_ref[...].astype(_BF)
        o_ref[...] = _glu(x, ws1_ref, ws2_ref)

    lo, hi = _expert_range(b, gs_ref)
    e = j - 1

    @pl.when((j > 0) & (e >= lo) & (e <= hi))
    def _routed():
        st = gs_ref[e + 1 - 1]
        en = gs_ref[e + 1]
        row = b * BM_M + jax.lax.broadcasted_iota(jnp.int32, (BM_M, 1), 0)
        msk = (row >= st) & (row < en)
        scale = jnp.where(msk, iw_ref[...], 0.0)
        x = (xp_ref[...] * scale).astype(_BF)
        y = _glu(x, w1_ref.at[0], w2_ref.at[0])
        o_ref[...] += y


_SC_NW = 32
_SC_BPW = T // _SC_NW


def _sc_gather_hw(t1, t2, idx):
    mesh = plsc.VectorSubcoreMesh(core_axis_name="c", subcore_axis_name="s")

    @functools.partial(
        pl.kernel, mesh=mesh,
        out_type=[jax.ShapeDtypeStruct((T, D), _F32),
                  jax.ShapeDtypeStruct((T, 16), _F32)],
        scratch_types=[pltpu.VMEM((_SC_BPW,), jnp.int32),
                       pltpu.VMEM((_SC_BPW, D), _F32),
                       pltpu.VMEM((_SC_BPW, 16), _F32),
                       pltpu.SemaphoreType.DMA],
    )
    def k(t1_hbm, t2_hbm, idx_hbm, o1_hbm, o2_hbm, idx_v, rows_v, w_v, sem):
        wid = jax.lax.axis_index("s") * 2 + jax.lax.axis_index("c")
        base = wid * _SC_BPW
        pltpu.sync_copy(idx_hbm.at[pl.ds(base, _SC_BPW)], idx_v)
        pltpu.async_copy(t1_hbm.at[idx_v], rows_v, sem).wait()
        pltpu.sync_copy(rows_v, o1_hbm.at[pl.ds(base, _SC_BPW)])
        pltpu.async_copy(t2_hbm.at[idx_v], w_v, sem).wait()
        pltpu.sync_copy(w_v, o2_hbm.at[pl.ds(base, _SC_BPW)])

    return k(t1, t2, idx)


def _sc_gather1(t1, idx):
    mesh = plsc.VectorSubcoreMesh(core_axis_name="c", subcore_axis_name="s")

    @functools.partial(
        pl.kernel, mesh=mesh,
        out_type=jax.ShapeDtypeStruct((T, D), _F32),
        scratch_types=[pltpu.VMEM((_SC_BPW,), jnp.int32),
                       pltpu.VMEM((_SC_BPW, D), _F32),
                       pltpu.SemaphoreType.DMA],
    )
    def k(t1_hbm, idx_hbm, o1_hbm, idx_v, rows_v, sem):
        wid = jax.lax.axis_index("s") * 2 + jax.lax.axis_index("c")
        base = wid * _SC_BPW
        pltpu.sync_copy(idx_hbm.at[pl.ds(base, _SC_BPW)], idx_v)
        pltpu.async_copy(t1_hbm.at[idx_v], rows_v, sem).wait()
        pltpu.sync_copy(rows_v, o1_hbm.at[pl.ds(base, _SC_BPW)])

    return k(t1, idx)


def kernel(hidden_states, cos, sin, w_in, wqkv, qk_w, wo, w_post,
           w_router, w1, w2, ws1, ws2):
    x = hidden_states.reshape(T, D)

    cosp = jnp.repeat(cos[:, : HD // 2], 2, axis=1)
    sinp = jnp.repeat(sin[:, : HD // 2], 2, axis=1)
    wcat = wqkv.astype(_BF)
    qkw_t = jnp.tile(qk_w, H + KVH).reshape(1, QKW)
    win2 = w_in.reshape(1, D)

    q, k, v = pl.pallas_call(
        _preattn_kernel,
        grid=(T // BM_A,),
        in_specs=[
            pl.BlockSpec((BM_A, D), lambda i: (i, 0)),
            pl.BlockSpec((D, QKVW), lambda i: (0, 0)),
            pl.BlockSpec((BM_A, HD), lambda i: (i, 0)),
            pl.BlockSpec((BM_A, HD), lambda i: (i, 0)),
            pl.BlockSpec((1, D), lambda i: (0, 0)),
            pl.BlockSpec((1, QKW), lambda i: (0, 0)),
        ],
        out_specs=[
            pl.BlockSpec((BM_A, H * HD), lambda i: (i, 0)),
            pl.BlockSpec((BM_A, KVH * HD), lambda i: (i, 0)),
            pl.BlockSpec((BM_A, KVH * HD), lambda i: (i, 0)),
        ],
        out_shape=[
            jax.ShapeDtypeStruct((T, H * HD), _BF),
            jax.ShapeDtypeStruct((T, KVH * HD), _BF),
            jax.ShapeDtypeStruct((T, KVH * HD), _BF),
        ],
    )(x, wcat, cosp, sinp, win2, qkw_t)

    GQ = H // KVH
    q4 = q.reshape(T, KVH, GQ, HD).transpose(1, 2, 0, 3)
    k3 = k.reshape(T, KVH, HD).transpose(1, 0, 2)
    v3 = jnp.concatenate(
        [v.reshape(T, KVH, HD), jnp.ones((T, KVH, 1), _BF)],
        axis=-1).transpose(1, 0, 2)
    attn4 = pl.pallas_call(
        _attn_kernel,
        grid=(KVH, T // BQ),
        in_specs=[
            pl.BlockSpec((1, GQ, BQ, HD), lambda g, i: (g, 0, i, 0)),
            pl.BlockSpec((1, T, HD), lambda g, i: (g, 0, 0)),
            pl.BlockSpec((1, T, HD + 1), lambda g, i: (g, 0, 0)),
        ],
        out_specs=pl.BlockSpec((1, GQ, BQ, HD), lambda g, i: (g, 0, i, 0)),
        out_shape=jax.ShapeDtypeStruct((KVH, GQ, T, HD), _BF),
    )(q4, k3, v3)
    attn = attn4.transpose(2, 0, 1, 3).reshape(T, H * HD)

    resid, h2, ids2, iw2 = pl.pallas_call(
        _postattn_kernel,
        grid=(T // BM_C,),
        in_specs=[
            pl.BlockSpec((BM_C, H * HD), lambda i: (i, 0)),
            pl.BlockSpec((H * HD, D), lambda i: (0, 0)),
            pl.BlockSpec((BM_C, D), lambda i: (i, 0)),
            pl.BlockSpec((1, D), lambda i: (0, 0)),
            pl.BlockSpec((D, E), lambda i: (0, 0)),
        ],
        out_specs=[
            pl.BlockSpec((BM_C, D), lambda i: (i, 0)),
            pl.BlockSpec((BM_C, D), lambda i: (i, 0)),
            pl.BlockSpec((BM_C, 1), lambda i: (i, 0)),
            pl.BlockSpec((BM_C, 1), lambda i: (i, 0)),
        ],
        out_shape=[
            jax.ShapeDtypeStruct((T, D), _F32),
            jax.ShapeDtypeStruct((T, D), _F32),
            jax.ShapeDtypeStruct((T, 1), jnp.int32),
            jax.ShapeDtypeStruct((T, 1), _F32),
        ],
    )(attn, wo.astype(_BF), x, w_post.reshape(1, D), w_router.T)

    ids = ids2[:, 0]
    order = jnp.argsort(ids).astype(jnp.int32)
    oh = (ids[:, None] == jnp.arange(E, dtype=ids.dtype)).astype(jnp.int32)
    cum = jnp.cumsum(oh, axis=0)
    gs = jnp.concatenate([jnp.zeros((1,), jnp.int32),
                          jnp.cumsum(cum[-1])]).astype(jnp.int32)
    rank = jnp.take_along_axis(cum, ids[:, None], axis=1)[:, 0] - 1
    pos = (jnp.take(gs, ids) + rank).astype(jnp.int32)
    h2s = _sc_gather1(h2, order)
    iws = jnp.take(iw2, order, axis=0)

    out_sorted = pl.pallas_call(
        _moe_kernel,
        grid_spec=pltpu.PrefetchScalarGridSpec(
            num_scalar_prefetch=1,
            grid=(NB_M, E + 1),
            in_specs=[
                pl.BlockSpec((BM_M, D), lambda b, j, gs_ref: (b, 0)),
                pl.BlockSpec((BM_M, 1), lambda b, j, gs_ref: (b, 0)),
                pl.BlockSpec((1, D, 2 * FF), _moe_widx),
                pl.BlockSpec((1, FF, D), _moe_widx),
                pl.BlockSpec((D, 2 * FF), lambda b, j, gs_ref: (0, 0)),
                pl.BlockSpec((FF, D), lambda b, j, gs_ref: (0, 0)),
            ],
            out_specs=pl.BlockSpec((BM_M, D), lambda b, j, gs_ref: (b, 0)),
        ),
        out_shape=jax.ShapeDtypeStruct((T, D), _F32),
    )(gs, h2s, iws, w1, w2, ws1.astype(_BF), ws2.astype(_BF))

    final = resid + _sc_gather1(out_sorted, pos)
    return final.reshape(B, S, D)


# speedup vs baseline: 1.0160x; 1.0160x over previous
import functools

import jax
import jax.numpy as jnp
from jax.experimental import pallas as pl
from jax.experimental.pallas import tpu as pltpu
from jax.experimental.pallas import tpu_sc as plsc

B, S, D = 1, 2048, 1024
H, KVH, HD = 16, 4, 64
E, FF = 8, 1024
T = B * S
QKW = (H + KVH) * HD
QKVW = (H + 2 * KVH) * HD

BM_A = 256
BQ = 512
BM_C = 256
BM_M = 512
NB_M = T // BM_M

_BF = jnp.bfloat16
_F32 = jnp.float32


def _preattn_kernel(x_ref, w_ref, cos_ref, sin_ref, win_ref, qkw_ref,
                    q_ref, k_ref, v_ref):
    x = x_ref[...]
    h = x * jax.lax.rsqrt(jnp.mean(x * x, axis=-1, keepdims=True) + 1e-5)
    h = h * win_ref[...]
    qkv = jnp.dot(h.astype(_BF), w_ref[...], preferred_element_type=_F32)
    base = qkv[:, :QKW]
    vpart = qkv[:, QKW:QKVW]
    lane = jax.lax.broadcasted_iota(jnp.int32, (BM_A, QKW), 1)
    rot = jnp.where(lane % 2 == 0,
                    -pltpu.roll(base, QKW - 1, 1),
                    pltpu.roll(base, 1, 1))
    cosT = jnp.concatenate([cos_ref[...]] * (H + KVH), axis=1)
    sinT = jnp.concatenate([sin_ref[...]] * (H + KVH), axis=1)
    qk = base * cosT + rot * sinT
    lane = jax.lax.broadcasted_iota(jnp.int32, (QKW, H + KVH), 0)
    head = jax.lax.broadcasted_iota(jnp.int32, (QKW, H + KVH), 1)
    G = (lane // HD == head).astype(_F32)
    laneT = jax.lax.broadcasted_iota(jnp.int32, (H + KVH, QKW), 1)
    headT = jax.lax.broadcasted_iota(jnp.int32, (H + KVH, QKW), 0)
    GT = (laneT // HD == headT).astype(_F32)
    ms = jnp.dot(qk * qk, G, preferred_element_type=_F32) / HD
    bc = jnp.dot(jax.lax.rsqrt(ms + 1e-6), GT, preferred_element_type=_F32)
    qkn = qk * bc * qkw_ref[...]
    q_ref[...] = (qkn[:, : H * HD] * (HD ** -0.5)).astype(_BF)
    k_ref[...] = qkn[:, H * HD:].astype(_BF)
    v_ref[...] = vpart.astype(_BF)


def _attn_post_kernel(q_ref, k_ref, v_ref, wo_ref, x_ref, wpost_ref,
                      wrt_ref, res_ref, h2_ref, ids_ref, iw_ref):
    i = pl.program_id(0)
    g = pl.program_id(1)
    GQ = H // KVH
    q = q_ref[0].reshape(GQ * BQ, HD)

    def chunk(j, acc, masked):
        kc = k_ref[0, pl.ds(j * BQ, BQ), :]
        s = jax.lax.dot_general(q, kc, (((1,), (1,)), ((), ())),
                                preferred_element_type=_F32)
        if masked:
            r = jax.lax.broadcasted_iota(jnp.int32, (GQ * BQ, BQ), 0)
            c = jax.lax.broadcasted_iota(jnp.int32, (GQ * BQ, BQ), 1)
            s = jnp.where(c <= r % BQ, s, -1e30)
        p = jnp.exp(s).astype(_BF)
        vc = v_ref[0, pl.ds(j * BQ, BQ), :]
        return acc + jnp.dot(p, vc, preferred_element_type=_F32)

    init = jnp.zeros((GQ * BQ, HD + 1), _F32)
    acc = jax.lax.fori_loop(0, i, lambda j, c: chunk(j, c, False), init)
    acc = chunk(i, acc, True)
    outb = (acc[:, :HD] / acc[:, HD:HD + 1]).astype(_BF)

    y = jnp.zeros((BQ, D), _F32)
    for gq in range(GQ):
        rows = outb[gq * BQ:(gq + 1) * BQ]
        wsl = wo_ref[pl.ds((g * GQ + gq) * HD, HD), :]
        y = y + jnp.dot(rows, wsl, preferred_element_type=_F32)

    @pl.when(g == 0)
    def _first():
        res_ref[...] = x_ref[...] + y

    @pl.when(g > 0)
    def _acc():
        res_ref[...] += y

    @pl.when(g == KVH - 1)
    def _tail():
        resid = res_ref[...]
        h2 = resid * jax.lax.rsqrt(
            jnp.mean(resid * resid, axis=-1, keepdims=True) + 1e-5)
        h2 = h2 * wpost_ref[...]
        h2_ref[...] = h2
        logits = jnp.dot(h2, wrt_ref[...], preferred_element_type=_F32)
        mx = jnp.max(logits, axis=-1, keepdims=True)
        e_iota = jax.lax.broadcasted_iota(jnp.int32, logits.shape, 1)
        ids_ref[...] = jnp.min(jnp.where(logits == mx, e_iota, E),
                               axis=-1, keepdims=True)
        iw_ref[...] = jax.nn.sigmoid(mx)


def _expert_range(b, gs_ref):
    start = b * BM_M
    end = start + BM_M
    lo = jnp.zeros((), jnp.int32)
    cnt = jnp.zeros((), jnp.int32)
    for e in range(E):
        lo = lo + (gs_ref[e + 1] <= start).astype(jnp.int32)
        cnt = cnt + (gs_ref[e] < end).astype(jnp.int32)
    return lo, cnt - 1


def _moe_widx(b, j, gs_ref):
    lo, hi = _expert_range(b, gs_ref)
    return (jnp.clip(j - 1, lo, hi), 0, 0)


_FCH = 2


def _glu(x, w1_ref, w2_ref):
    fc = FF // _FCH
    y = jnp.zeros((x.shape[0], D), _F32)
    for f in range(_FCH):
        g = jnp.dot(x, w1_ref[:, f * fc:(f + 1) * fc].astype(_BF),
                    preferred_element_type=_F32)
        u = jnp.dot(x, w1_ref[:, FF + f * fc:FF + (f + 1) * fc].astype(_BF),
                    preferred_element_type=_F32)
        act = (jax.nn.silu(g) * u).astype(_BF)
        y = y + jnp.dot(act, w2_ref[f * fc:(f + 1) * fc, :].astype(_BF),
                        preferred_element_type=_F32)
    return y


def _moe_kernel(gs_ref, xp_ref, iw_ref, w1_ref, w2_ref,
                ws1_ref, ws2_ref, o_ref):
    b = pl.program_id(0)
    j = pl.program_id(1)

    @pl.when(j == 0)
    def _shared():
        x = xp_ref[...].astype(_BF)
        o_ref[...] = _glu(x, ws1_ref, ws2_ref)

    lo, hi = _expert_range(b, gs_ref)
    e = j - 1

    @pl.when((j > 0) & (e >= lo) & (e <= hi))
    def _routed():
        st = gs_ref[e + 1 - 1]
        en = gs_ref[e + 1]
        row = b * BM_M + jax.lax.broadcasted_iota(jnp.int32, (BM_M, 1), 0)
        msk = (row >= st) & (row < en)
        scale = jnp.where(msk, iw_ref[...], 0.0)
        x = (xp_ref[...] * scale).astype(_BF)
        y = _glu(x, w1_ref.at[0], w2_ref.at[0])
        o_ref[...] += y


_SC_NW = 32
_SC_BPW = T // _SC_NW


def _sc_gather_hw(t1, t2, idx):
    mesh = plsc.VectorSubcoreMesh(core_axis_name="c", subcore_axis_name="s")

    @functools.partial(
        pl.kernel, mesh=mesh,
        out_type=[jax.ShapeDtypeStruct((T, D), _F32),
                  jax.ShapeDtypeStruct((T, 16), _F32)],
        scratch_types=[pltpu.VMEM((_SC_BPW,), jnp.int32),
                       pltpu.VMEM((_SC_BPW, D), _F32),
                       pltpu.VMEM((_SC_BPW, 16), _F32),
                       pltpu.SemaphoreType.DMA],
    )
    def k(t1_hbm, t2_hbm, idx_hbm, o1_hbm, o2_hbm, idx_v, rows_v, w_v, sem):
        wid = jax.lax.axis_index("s") * 2 + jax.lax.axis_index("c")
        base = wid * _SC_BPW
        pltpu.sync_copy(idx_hbm.at[pl.ds(base, _SC_BPW)], idx_v)
        pltpu.async_copy(t1_hbm.at[idx_v], rows_v, sem).wait()
        pltpu.sync_copy(rows_v, o1_hbm.at[pl.ds(base, _SC_BPW)])
        pltpu.async_copy(t2_hbm.at[idx_v], w_v, sem).wait()
        pltpu.sync_copy(w_v, o2_hbm.at[pl.ds(base, _SC_BPW)])

    return k(t1, t2, idx)


def _sc_gather1(t1, idx):
    mesh = plsc.VectorSubcoreMesh(core_axis_name="c", subcore_axis_name="s")

    @functools.partial(
        pl.kernel, mesh=mesh,
        out_type=jax.ShapeDtypeStruct((T, D), _F32),
        scratch_types=[pltpu.VMEM((_SC_BPW,), jnp.int32),
                       pltpu.VMEM((_SC_BPW, D), _F32),
                       pltpu.SemaphoreType.DMA],
    )
    def k(t1_hbm, idx_hbm, o1_hbm, idx_v, rows_v, sem):
        wid = jax.lax.axis_index("s") * 2 + jax.lax.axis_index("c")
        base = wid * _SC_BPW
        pltpu.sync_copy(idx_hbm.at[pl.ds(base, _SC_BPW)], idx_v)
        pltpu.async_copy(t1_hbm.at[idx_v], rows_v, sem).wait()
        pltpu.sync_copy(rows_v, o1_hbm.at[pl.ds(base, _SC_BPW)])

    return k(t1, idx)


def kernel(hidden_states, cos, sin, w_in, wqkv, qk_w, wo, w_post,
           w_router, w1, w2, ws1, ws2):
    x = hidden_states.reshape(T, D)

    cosp = jnp.repeat(cos[:, : HD // 2], 2, axis=1)
    sinp = jnp.repeat(sin[:, : HD // 2], 2, axis=1)
    wcat = wqkv.astype(_BF)
    qkw_t = jnp.tile(qk_w, H + KVH).reshape(1, QKW)
    win2 = w_in.reshape(1, D)

    q, k, v = pl.pallas_call(
        _preattn_kernel,
        grid=(T // BM_A,),
        in_specs=[
            pl.BlockSpec((BM_A, D), lambda i: (i, 0)),
            pl.BlockSpec((D, QKVW), lambda i: (0, 0)),
            pl.BlockSpec((BM_A, HD), lambda i: (i, 0)),
            pl.BlockSpec((BM_A, HD), lambda i: (i, 0)),
            pl.BlockSpec((1, D), lambda i: (0, 0)),
            pl.BlockSpec((1, QKW), lambda i: (0, 0)),
        ],
        out_specs=[
            pl.BlockSpec((BM_A, H * HD), lambda i: (i, 0)),
            pl.BlockSpec((BM_A, KVH * HD), lambda i: (i, 0)),
            pl.BlockSpec((BM_A, KVH * HD), lambda i: (i, 0)),
        ],
        out_shape=[
            jax.ShapeDtypeStruct((T, H * HD), _BF),
            jax.ShapeDtypeStruct((T, KVH * HD), _BF),
            jax.ShapeDtypeStruct((T, KVH * HD), _BF),
        ],
    )(x, wcat, cosp, sinp, win2, qkw_t)

    GQ = H // KVH
    q4 = q.reshape(T, KVH, GQ, HD).transpose(1, 2, 0, 3)
    k3 = k.reshape(T, KVH, HD).transpose(1, 0, 2)
    v3 = jnp.concatenate(
        [v.reshape(T, KVH, HD), jnp.ones((T, KVH, 1), _BF)],
        axis=-1).transpose(1, 0, 2)
    resid, h2, ids2, iw2 = pl.pallas_call(
        _attn_post_kernel,
        grid=(T // BQ, KVH),
        in_specs=[
            pl.BlockSpec((1, GQ, BQ, HD), lambda i, g: (g, 0, i, 0)),
            pl.BlockSpec((1, T, HD), lambda i, g: (g, 0, 0)),
            pl.BlockSpec((1, T, HD + 1), lambda i, g: (g, 0, 0)),
            pl.BlockSpec((H * HD, D), lambda i, g: (0, 0)),
            pl.BlockSpec((BQ, D), lambda i, g: (i, 0)),
            pl.BlockSpec((1, D), lambda i, g: (0, 0)),
            pl.BlockSpec((D, E), lambda i, g: (0, 0)),
        ],
        out_specs=[
            pl.BlockSpec((BQ, D), lambda i, g: (i, 0)),
            pl.BlockSpec((BQ, D), lambda i, g: (i, 0)),
            pl.BlockSpec((BQ, 1), lambda i, g: (i, 0)),
            pl.BlockSpec((BQ, 1), lambda i, g: (i, 0)),
        ],
        out_shape=[
            jax.ShapeDtypeStruct((T, D), _F32),
            jax.ShapeDtypeStruct((T, D), _F32),
            jax.ShapeDtypeStruct((T, 1), jnp.int32),
            jax.ShapeDtypeStruct((T, 1), _F32),
        ],
    )(q4, k3, v3, wo.astype(_BF), x, w_post.reshape(1, D), w_router.T)

    ids = ids2[:, 0]
    order = jnp.argsort(ids).astype(jnp.int32)
    ids_sorted = jnp.take(ids, order)
    gs = jnp.searchsorted(ids_sorted, jnp.arange(E + 1, dtype=jnp.int32),
                          side='left').astype(jnp.int32)
    h2s = _sc_gather1(h2, order)
    iws = jnp.take(iw2, order, axis=0)

    out_sorted = pl.pallas_call(
        _moe_kernel,
        grid_spec=pltpu.PrefetchScalarGridSpec(
            num_scalar_prefetch=1,
            grid=(NB_M, E + 1),
            in_specs=[
                pl.BlockSpec((BM_M, D), lambda b, j, gs_ref: (b, 0)),
                pl.BlockSpec((BM_M, 1), lambda b, j, gs_ref: (b, 0)),
                pl.BlockSpec((1, D, 2 * FF), _moe_widx),
                pl.BlockSpec((1, FF, D), _moe_widx),
                pl.BlockSpec((D, 2 * FF), lambda b, j, gs_ref: (0, 0)),
                pl.BlockSpec((FF, D), lambda b, j, gs_ref: (0, 0)),
            ],
            out_specs=pl.BlockSpec((BM_M, D), lambda b, j, gs_ref: (b, 0)),
        ),
        out_shape=jax.ShapeDtypeStruct((T, D), _F32),
    )(gs, h2s, iws, w1, w2, ws1.astype(_BF), ws2.astype(_BF))

    inv = jnp.argsort(order).astype(jnp.int32)
    final = resid + _sc_gather1(out_sorted, inv)
    return final.reshape(B, S, D)
